# Initial kernel scaffold; baseline (speedup 1.0000x reference)
import jax
import jax.numpy as jnp
from jax.experimental import pallas as pl


def kernel(tokens, W):
    raise NotImplementedError("write your pallas kernel here")


# speedup vs baseline: 4.2637x; 4.2637x over previous
import functools
import math

import jax
import jax.numpy as jnp
from jax import lax
from jax.experimental import pallas as pl
from jax.experimental.pallas import tpu as pltpu
from jax.experimental.pallas import tpu_sc as plsc

D_MODEL = 64
SCALE = math.sqrt(D_MODEL)
CHUNK = 128
NBUF = 4


@functools.partial(jax.jit, static_argnames=("batch",))
def _embed(tokens_flat, W, *, batch):
  info = plsc.get_sparse_core_info()
  nc, ns, nl = info.num_cores, info.num_subcores, info.num_lanes
  nw = nc * ns
  b_per_w = batch // nw
  nchunk = b_per_w // CHUNK
  ngroup = nchunk // NBUF
  d = D_MODEL

  mesh = plsc.VectorSubcoreMesh(core_axis_name="c", subcore_axis_name="s")

  @functools.partial(
      pl.kernel,
      out_type=jax.ShapeDtypeStruct((batch, d), jnp.float32),
      mesh=mesh,
      compiler_params=pltpu.CompilerParams(use_tc_tiling_on_sc=False),
      scratch_types=(
          [pltpu.VMEM((b_per_w,), jnp.int32)]
          + [pltpu.VMEM((CHUNK, d), jnp.float32) for _ in range(2 * NBUF)]
          + [pltpu.SemaphoreType.DMA for _ in range(2 * NBUF)]
      ),
  )
  def k(tok_hbm, w_hbm, out_hbm, idx_v, *bufs_and_sems):
    in_bufs = bufs_and_sems[:NBUF]
    out_bufs = bufs_and_sems[NBUF:2 * NBUF]
    g_sems = bufs_and_sems[2 * NBUF:3 * NBUF]
    s_sems = bufs_and_sems[3 * NBUF:]

    wid = lax.axis_index("s") * nc + lax.axis_index("c")
    base = wid * b_per_w

    pltpu.sync_copy(tok_hbm.at[pl.ds(base, b_per_w)], idx_v)

    def fire_gather(j, b):
      pltpu.async_copy(
          w_hbm.at[idx_v.at[pl.ds(j * CHUNK, CHUNK)]], in_bufs[b], g_sems[b])

    def wait_gather(b):
      pltpu.make_async_copy(
          w_hbm.at[pl.ds(0, CHUNK)], in_bufs[b], g_sems[b]).wait()

    def fire_scatter(j, b):
      pltpu.async_copy(
          out_bufs[b], out_hbm.at[pl.ds(base + j * CHUNK, CHUNK)], s_sems[b])

    def wait_scatter(b):
      pltpu.make_async_copy(
          out_bufs[b], out_hbm.at[pl.ds(base, CHUNK)], s_sems[b]).wait()

    for b in range(NBUF):
      fire_gather(b, b)

    @pl.loop(0, ngroup)
    def _group(g):
      for b in range(NBUF):
        j = g * NBUF + b
        wait_gather(b)

        @pl.when(g > 0)
        def _():
          wait_scatter(b)

        @plsc.parallel_loop(0, CHUNK, unroll=2)
        def _scale(r):
          for c in range(d // nl):
            out_bufs[b][r, pl.ds(c * nl, nl)] = (
                in_bufs[b][r, pl.ds(c * nl, nl)] * SCALE)

        fire_scatter(j, b)

        @pl.when(g < ngroup - 1)
        def _():
          fire_gather(j + NBUF, b)

    for b in range(NBUF):
      wait_scatter(b)

  return k(tokens_flat, W)


def kernel(tokens, W):
  shape = tokens.shape
  batch = shape[0] * shape[1]
  tokens_flat = tokens.reshape(-1).astype(jnp.int32)
  out = _embed(tokens_flat, W, batch=batch)
  return out.reshape(shape[0], shape[1], D_MODEL)


# speedup vs baseline: 5.5798x; 1.3087x over previous
import functools
import math

import jax
import jax.numpy as jnp
from jax import lax
from jax.experimental import pallas as pl
from jax.experimental.pallas import tpu as pltpu
from jax.experimental.pallas import tpu_sc as plsc

D_MODEL = 64
D_PAD = 128
SCALE = math.sqrt(D_MODEL)
CHUNK = 128
NBUF = 3


@functools.partial(jax.jit, static_argnames=("batch",))
def _embed(tokens_flat, W_padded, *, batch):
  info = plsc.get_sparse_core_info()
  nc, ns, nl = info.num_cores, info.num_subcores, info.num_lanes
  nw = nc * ns
  b_per_w = batch // nw
  nchunk = b_per_w // CHUNK
  ngroup = nchunk // NBUF
  d = D_MODEL

  mesh = plsc.VectorSubcoreMesh(core_axis_name="c", subcore_axis_name="s")

  @functools.partial(
      pl.kernel,
      out_type=jax.ShapeDtypeStruct((batch, d), jnp.float32),
      mesh=mesh,
      scratch_types=(
          [pltpu.VMEM((b_per_w,), jnp.int32)]
          + [pltpu.VMEM((CHUNK, D_PAD), jnp.float32) for _ in range(NBUF)]
          + [pltpu.VMEM((CHUNK, d), jnp.float32) for _ in range(NBUF)]
          + [pltpu.SemaphoreType.DMA for _ in range(2 * NBUF)]
      ),
  )
  def k(tok_hbm, w_hbm, out_hbm, idx_v, *bufs_and_sems):
    in_bufs = bufs_and_sems[:NBUF]
    out_bufs = bufs_and_sems[NBUF:2 * NBUF]
    g_sems = bufs_and_sems[2 * NBUF:3 * NBUF]
    s_sems = bufs_and_sems[3 * NBUF:]

    wid = lax.axis_index("s") * nc + lax.axis_index("c")
    base = wid * b_per_w

    pltpu.sync_copy(tok_hbm.at[pl.ds(base, b_per_w)], idx_v)

    def fire_gather(j, b):
      pltpu.async_copy(
          w_hbm.at[idx_v.at[pl.ds(j * CHUNK, CHUNK)]], in_bufs[b], g_sems[b])

    def wait_gather(b):
      pltpu.make_async_copy(
          w_hbm.at[pl.ds(0, CHUNK)], in_bufs[b], g_sems[b]).wait()

    def fire_scatter(j, b):
      pltpu.async_copy(
          out_bufs[b], out_hbm.at[pl.ds(base + j * CHUNK, CHUNK)], s_sems[b])

    def wait_scatter(b):
      pltpu.make_async_copy(
          out_bufs[b], out_hbm.at[pl.ds(base, CHUNK)], s_sems[b]).wait()

    def do_scale(b):
      @plsc.parallel_loop(0, CHUNK, unroll=2)
      def _scale(r):
        for c in range(d // nl):
          out_bufs[b][r, pl.ds(c * nl, nl)] = (
              in_bufs[b][r, pl.ds(c * nl, nl)] * SCALE)

    for b in range(NBUF):
      fire_gather(b, b)

    @pl.loop(0, ngroup)
    def _group(g):
      for b in range(NBUF):
        j = g * NBUF + b
        wait_gather(b)

        @pl.when(j >= NBUF)
        def _():
          wait_scatter(b)

        do_scale(b)
        fire_scatter(j, b)

        @pl.when(j + NBUF < nchunk)
        def _():
          fire_gather(j + NBUF, b)

    for r in range(nchunk - ngroup * NBUF):
      j = ngroup * NBUF + r
      wait_gather(r)
      wait_scatter(r)
      do_scale(r)
      fire_scatter(j, r)

    for b in range(NBUF):
      wait_scatter(b)

  return k(tokens_flat, W_padded)


def kernel(tokens, W):
  shape = tokens.shape
  batch = shape[0] * shape[1]
  tokens_flat = tokens.reshape(-1).astype(jnp.int32)
  W_padded = jnp.pad(W, ((0, 0), (0, D_PAD - D_MODEL)))
  out = _embed(tokens_flat, W_padded, batch=batch)
  return out.reshape(shape[0], shape[1], D_MODEL)
